# baseline (speedup 1.0000x reference)
import functools

import jax
import jax.numpy as jnp
from jax import lax
from jax.experimental import pallas as pl
from jax.experimental.pallas import tpu as pltpu
from jax.experimental.pallas import tpu_sc as plsc


@functools.cache
def _build(V, D, R, S):
    info = plsc.get_sparse_core_info()
    NC, NS = info.num_cores, info.num_subcores
    NW = NC * NS
    assert R % NW == 0
    r_per_w = R // NW
    CR = 8
    SP = -(-S // 16) * 16
    assert r_per_w % (2 * CR) == 0
    n_chunks = r_per_w // CR

    mesh = plsc.VectorSubcoreMesh(core_axis_name="c", subcore_axis_name="s")

    @functools.partial(
        pl.kernel,
        mesh=mesh,
        compiler_params=pltpu.CompilerParams(use_tc_tiling_on_sc=False),
        out_type=jax.ShapeDtypeStruct((R, S, D), jnp.float32),
        scratch_types=[
            pltpu.VMEM((CR, SP), jnp.int32),
            pltpu.VMEM((CR, SP), jnp.int32),
            pltpu.VMEM((CR, SP, D), jnp.float32),
            pltpu.VMEM((CR, SP, D), jnp.float32),
            pltpu.SemaphoreType.DMA,
            pltpu.SemaphoreType.DMA,
            pltpu.SemaphoreType.DMA,
            pltpu.SemaphoreType.DMA,
        ],
    )
    def gather_kernel(idx_hbm, table_hbm, out_hbm, idx0, idx1, rows0, rows1,
                      gsem0, gsem1, ssem0, ssem1):
        wid = lax.axis_index("s") * NC + lax.axis_index("c")
        base_row = wid * r_per_w

        def load_idx(c, idx_v):
            pltpu.sync_copy(
                idx_hbm.at[pl.ds(base_row + c * CR, CR), :], idx_v)

        def g_desc(idx_v, rows, gsem, r):
            return pltpu.make_async_copy(
                table_hbm.at[idx_v.at[r]], rows.at[r], gsem)

        def g_start(idx_v, rows, gsem):
            for r in range(CR):
                g_desc(idx_v, rows, gsem, r).start()

        def g_wait(idx_v, rows, gsem):
            for r in range(CR):
                g_desc(idx_v, rows, gsem, r).wait()

        def s_desc(c, rows, ssem):
            return pltpu.make_async_copy(
                rows.at[:, pl.ds(0, S), :],
                out_hbm.at[pl.ds(base_row + c * CR, CR), :, :], ssem)

        load_idx(0, idx0)
        g_start(idx0, rows0, gsem0)
        load_idx(1, idx1)
        g_start(idx1, rows1, gsem1)

        bufs = ((idx0, rows0, gsem0, ssem0), (idx1, rows1, gsem1, ssem1))

        def body(g2, carry):
            g = g2 * 2
            for b in range(2):
                c = g + b
                idx_v, rows, gsem, ssem = bufs[b]
                g_wait(idx_v, rows, gsem)
                s_desc(c, rows, ssem).start()

                @pl.when(c + 2 < n_chunks)
                def _():
                    s_desc(c, rows, ssem).wait()
                    load_idx(c + 2, idx_v)
                    g_start(idx_v, rows, gsem)

            return carry

        lax.fori_loop(0, n_chunks // 2, body, 0)
        s_desc(n_chunks - 2, rows0, ssem0).wait()
        s_desc(n_chunks - 1, rows1, ssem1).wait()

    return gather_kernel


def kernel(data, in_table, out_table):
    R, S = data.shape
    V, D = in_table.shape
    SP = -(-S // 16) * 16
    idx = jnp.pad(data.astype(jnp.int32), ((0, 0), (0, SP - S)))
    return _build(V, D, R, S)(idx, in_table)


# speedup vs baseline: 4.6759x; 4.6759x over previous
import functools

import jax
import jax.numpy as jnp
from jax import lax
from jax.experimental import pallas as pl
from jax.experimental.pallas import tpu as pltpu
from jax.experimental.pallas import tpu_sc as plsc


@functools.cache
def _build(V, D, R, S):
    info = plsc.get_sparse_core_info()
    NC, NS = info.num_cores, info.num_subcores
    NW = NC * NS
    assert R % NW == 0
    CB = R // NW
    assert S % 2 == 0

    mesh = plsc.VectorSubcoreMesh(core_axis_name="c", subcore_axis_name="s")

    @functools.partial(
        pl.kernel,
        mesh=mesh,
        compiler_params=pltpu.CompilerParams(use_tc_tiling_on_sc=False),
        out_type=jax.ShapeDtypeStruct((R, S, D), jnp.float32),
        scratch_types=[
            pltpu.VMEM((CB,), jnp.int32),
            pltpu.VMEM((CB,), jnp.int32),
            pltpu.VMEM((CB, D), jnp.float32),
            pltpu.VMEM((CB, D), jnp.float32),
            pltpu.SemaphoreType.DMA,
            pltpu.SemaphoreType.DMA,
            pltpu.SemaphoreType.DMA,
            pltpu.SemaphoreType.DMA,
        ],
    )
    def gather_kernel(idxt_hbm, table_hbm, out_hbm, idx0, idx1, rows0, rows1,
                      gsem0, gsem1, ssem0, ssem1):
        wid = lax.axis_index("s") * NC + lax.axis_index("c")
        b0 = wid * CB

        def load_idx(s, idx_v):
            pltpu.sync_copy(idxt_hbm.at[s, pl.ds(b0, CB)], idx_v)

        def g_desc(idx_v, rows, gsem):
            return pltpu.make_async_copy(table_hbm.at[idx_v], rows, gsem)

        def s_desc(s, rows, ssem):
            return pltpu.make_async_copy(
                rows, out_hbm.at[pl.ds(b0, CB), s, :], ssem)

        load_idx(0, idx0)
        g_desc(idx0, rows0, gsem0).start()
        load_idx(1, idx1)
        g_desc(idx1, rows1, gsem1).start()

        bufs = ((idx0, rows0, gsem0, ssem0), (idx1, rows1, gsem1, ssem1))

        def body(g2, carry):
            g = g2 * 2
            for b in range(2):
                s = g + b
                idx_v, rows, gsem, ssem = bufs[b]
                g_desc(idx_v, rows, gsem).wait()
                s_desc(s, rows, ssem).start()

                @pl.when(s + 2 < S)
                def _():
                    s_desc(s, rows, ssem).wait()
                    load_idx(s + 2, idx_v)
                    g_desc(idx_v, rows, gsem).start()

            return carry

        lax.fori_loop(0, S // 2, body, 0)
        s_desc(S - 2, rows0, ssem0).wait()
        s_desc(S - 1, rows1, ssem1).wait()

    return gather_kernel


def kernel(data, in_table, out_table):
    R, S = data.shape
    V, D = in_table.shape
    idxt = data.astype(jnp.int32).T
    return _build(V, D, R, S)(idxt, in_table)
